# baseline (speedup 1.0000x reference)
import functools

import jax
import jax.numpy as jnp
from jax import lax
from jax.experimental import pallas as pl
from jax.experimental.pallas import tpu as pltpu
from jax.experimental.pallas import tpu_sc as plsc

D = 128
H = D // 2
NC = 2
NS = 16
L = 16
CHUNK_K = 80
HI32 = -65536
RND = 32768


def _pack2(hi_f32, lo_f32):
    hb = jax.lax.bitcast_convert_type(hi_f32, jnp.int32) + RND
    lb = jax.lax.bitcast_convert_type(lo_f32, jnp.int32) + RND
    return (hb & HI32) | jax.lax.shift_right_logical(lb, 16)


def _dense1_body(x0_ref, w1t_ref, b1_ref, w2t_ref, b2_ref, a1wt_ref, a2wt_ref,
                 a1b_ref, a2b_ref, xjp_ref, sp_ref):
    x = x0_ref[...]
    xi = jnp.dot(x, w1t_ref[...], preferred_element_type=jnp.float32) + b1_ref[...]
    xi = jnp.where(xi > 0, xi, 0.2 * xi)
    xj = jnp.dot(x, w2t_ref[...], preferred_element_type=jnp.float32) + b2_ref[...]
    xj = jnp.where(xj > 0, xj, 0.2 * xj)
    xjp_ref[...] = _pack2(xj[:, :H], xj[:, H:])
    ai = jnp.sum(xi * a1wt_ref[...], axis=1, keepdims=True) + a1b_ref[0, 0]
    aj = jnp.sum(xj * a2wt_ref[...], axis=1, keepdims=True) + a2b_ref[0, 0]
    sp_ref[...] = _pack2(ai, aj)


def _dense1(x0, W1, b1, W2, b2, a1w, a1b, a2w, a2b, bn):
    n = x0.shape[0]
    grid = (n // bn,)
    full = lambda shape: pl.BlockSpec(shape, lambda i: (0, 0))
    return pl.pallas_call(
        _dense1_body,
        grid=grid,
        in_specs=[
            pl.BlockSpec((bn, D), lambda i: (i, 0)),
            full((D, D)), full((1, D)), full((D, D)), full((1, D)),
            full((1, D)), full((1, D)),
            pl.BlockSpec(memory_space=pltpu.SMEM),
            pl.BlockSpec(memory_space=pltpu.SMEM),
        ],
        out_specs=[
            pl.BlockSpec((bn, H), lambda i: (i, 0)),
            pl.BlockSpec((bn, 1), lambda i: (i, 0)),
        ],
        out_shape=[
            jax.ShapeDtypeStruct((n, H), jnp.int32),
            jax.ShapeDtypeStruct((n, 1), jnp.int32),
        ],
    )(x0, W1.T, b1.reshape(1, D), W2.T, b2.reshape(1, D),
      a1w.reshape(1, D), a2w.reshape(1, D),
      a1b.reshape(1, 1), a2b.reshape(1, 1))


def _dense2_body(p_ref, x0_ref, g1_ref, be1_ref, g2_ref, be2_ref, wot_ref,
                 bo_ref, out_ref):
    u = p_ref[0] + p_ref[1] + x0_ref[...]
    mu = jnp.mean(u, axis=1, keepdims=True)
    var = jnp.mean((u - mu) ** 2, axis=1, keepdims=True)
    y = g1_ref[...] * (u - mu) / jnp.sqrt(var + 1e-5) + be1_ref[...]
    v = jnp.dot(y, wot_ref[...], preferred_element_type=jnp.float32) + bo_ref[...] + y
    mu2 = jnp.mean(v, axis=1, keepdims=True)
    var2 = jnp.mean((v - mu2) ** 2, axis=1, keepdims=True)
    out_ref[...] = g2_ref[...] * (v - mu2) / jnp.sqrt(var2 + 1e-5) + be2_ref[...]


def _dense2(partials, x0, g1, be1, g2, be2, Wo, bo, bn):
    n = x0.shape[0]
    grid = (n // bn,)
    full = lambda shape: pl.BlockSpec(shape, lambda i: (0, 0))
    return pl.pallas_call(
        _dense2_body,
        grid=grid,
        in_specs=[
            pl.BlockSpec((2, bn, D), lambda i: (0, i, 0)),
            pl.BlockSpec((bn, D), lambda i: (i, 0)),
            full((1, D)), full((1, D)), full((1, D)), full((1, D)),
            full((D, D)), full((1, D)),
        ],
        out_specs=pl.BlockSpec((bn, D), lambda i: (i, 0)),
        out_shape=jax.ShapeDtypeStruct((n, D), jnp.float32),
    )(partials, x0, g1.reshape(1, D), be1.reshape(1, D), g2.reshape(1, D),
      be2.reshape(1, D), Wo.T, bo.reshape(1, D))


def _make_edge_kernel(n, e):
    nw = NC * NS
    assert e % nw == 0
    e_per_w = e // nw
    K = CHUNK_K
    assert e_per_w % K == 0
    n_chunks = e_per_w // K
    NGB = 3
    NRB = 2
    NIB = 6
    rpt = (n // NS) // 8 * 8
    tail = n - NS * rpt
    assert tail % 8 == 0
    ZQ = 48
    assert rpt % ZQ == 0 and tail <= ZQ and ZQ <= K

    mesh = plsc.VectorSubcoreMesh(core_axis_name="c", subcore_axis_name="s",
                                  num_cores=NC, num_subcores=NS)

    @functools.partial(
        pl.kernel,
        out_type=jax.ShapeDtypeStruct((NC, n, D), jnp.float32),
        mesh=mesh,
        scratch_types=[
            [pltpu.VMEM((2, K), jnp.int32) for _ in range(NIB)],
            [pltpu.VMEM((K, H), jnp.int32) for _ in range(NGB)],
            [pltpu.VMEM((K, D), jnp.float32) for _ in range(NRB)],
            pltpu.VMEM((K,), jnp.float32),
            pltpu.VMEM((n,), jnp.int32),
            pltpu.VMEM_SHARED((n, D), jnp.float32),
            [pltpu.SemaphoreType.DMA for _ in range(NIB)],
            [pltpu.SemaphoreType.DMA for _ in range(NGB)],
            [pltpu.SemaphoreType.DMA for _ in range(NRB)],
            pltpu.SemaphoreType.DMA,
        ],
        compiler_params=pltpu.CompilerParams(needs_layout_passes=False,
                                            use_tc_tiling_on_sc=False),
    )
    def edge_kernel(xjp_hbm, sp_hbm, rc_hbm, out_hbm,
                    idx, gbuf, rows, vals_v, spk_v, acc_sh, sI, sg, ss, sz):
        cid = lax.axis_index("c")
        sid = lax.axis_index("s")
        wid = cid * NS + sid

        zeros16 = jnp.zeros((L,), jnp.float32)
        for r in range(ZQ):
            for c in range(D // L):
                rows[0][r, pl.ds(c * L, L)] = zeros16

        def zfire(j, _):
            pltpu.async_copy(rows[0].at[pl.ds(0, ZQ)],
                             acc_sh.at[pl.ds(sid * rpt + j * ZQ, ZQ)], sz)
            return 0

        lax.fori_loop(0, rpt // ZQ, zfire, 0)

        @pl.when(sid == NS - 1)
        def _zero_tail():
            pltpu.async_copy(rows[0].at[pl.ds(0, tail)],
                             acc_sh.at[pl.ds(NS * rpt, tail)], sz)

        pltpu.sync_copy(sp_hbm, spk_v)

        def zdrain(j, _):
            pltpu.make_async_copy(rows[0].at[pl.ds(0, ZQ)],
                                  acc_sh.at[pl.ds(0, ZQ)], sz).wait()
            return 0

        lax.fori_loop(0, rpt // ZQ, zdrain, 0)

        @pl.when(sid == NS - 1)
        def _zero_tail_drain():
            pltpu.make_async_copy(rows[0].at[pl.ds(0, tail)],
                                  acc_sh.at[pl.ds(0, tail)], sz).wait()

        def idx_load(ch, s):
            pltpu.async_copy(rc_hbm.at[wid, ch], idx[s], sI[s])

        def idx_wait(s):
            pltpu.make_async_copy(rc_hbm.at[wid, 0], idx[s], sI[s]).wait()

        for s in range(4):
            idx_load(s, s)
        for s in range(2):
            idx_wait(s)

        plsc.subcore_barrier()

        for b in range(2):
            pltpu.async_copy(xjp_hbm.at[idx[b].at[1]], gbuf[b], sg[b])

        def step(j, t):
            p = t % NGB
            p2 = t % NRB
            q = (t + 2) % NGB
            s2 = (t + 2) % NIB
            s4 = (t + 4) % NIB
            static = isinstance(j, int)

            for g in range(K // L):
                r16 = idx[t][0, pl.ds(g * L, L)]
                c16 = idx[t][1, pl.ds(g * L, L)]
                br = plsc.load_gather(spk_v, [r16])
                bc = plsc.load_gather(spk_v, [c16])
                sc = plsc.bitcast(br & HI32, jnp.float32) + \
                    plsc.bitcast(bc << 16, jnp.float32)
                vals_v[pl.ds(g * L, L)] = 1.0 / (1.0 + jnp.exp(-sc))

            pltpu.make_async_copy(xjp_hbm.at[idx[t].at[1]], gbuf[p], sg[p]).wait()

            if not static or j + 2 < n_chunks:
                idx_wait(s2)
                pltpu.async_copy(xjp_hbm.at[idx[s2].at[1]], gbuf[q], sg[q])

            if not static or j >= NRB:
                pltpu.make_async_copy(rows[p2], acc_sh.at[idx[0].at[0]],
                                      ss[p2]).wait()

            @plsc.parallel_loop(0, K, unroll=4)
            def _scale(ei):
                sp = plsc.load_gather(vals_v, [jnp.full((L,), ei, jnp.int32)])
                for w in range(H // L):
                    wv = gbuf[p][ei, pl.ds(w * L, L)]
                    a = plsc.bitcast(wv & HI32, jnp.float32)
                    b = plsc.bitcast(wv << 16, jnp.float32)
                    rows[p2][ei, pl.ds(w * L, L)] = a * sp
                    rows[p2][ei, pl.ds(H + w * L, L)] = b * sp

            pltpu.async_copy(rows[p2], acc_sh.at[idx[t].at[0]], ss[p2], add=True)

            if not static or j + 4 < n_chunks:
                idx_load(j + 4, s4)

        for j in range(NIB):
            step(j, j)

        n_steady = n_chunks // NIB * NIB

        def steady(j6, _):
            for t in range(NIB):
                step(j6 * NIB + t, t)
            return 0

        lax.fori_loop(1, n_steady // NIB, steady, 0)

        for j in range(n_steady, n_chunks):
            step(j, j % NIB)

        for p2 in range(NRB):
            pltpu.make_async_copy(rows[p2], acc_sh.at[idx[0].at[0]],
                                  ss[p2]).wait()
        plsc.subcore_barrier()

        r0 = sid * rpt
        pltpu.sync_copy(acc_sh.at[pl.ds(r0, rpt)],
                        out_hbm.at[cid, pl.ds(r0, rpt)])

        @pl.when(sid == NS - 1)
        def _copy_tail():
            pltpu.sync_copy(acc_sh.at[pl.ds(NS * rpt, tail)],
                            out_hbm.at[cid, pl.ds(NS * rpt, tail)])

    return edge_kernel


def kernel(x0, x1, edge_index, W1, b1, W2, b2, a1w, a1b, a2w, a2b,
           g1, be1, g2, be2, Wo, bo):
    del x1
    n = x0.shape[0]
    e = edge_index.shape[1]
    ei = edge_index.astype(jnp.int32)
    nw = NC * NS
    rc = jnp.stack([ei[0].reshape(nw, -1, CHUNK_K),
                    ei[1].reshape(nw, -1, CHUNK_K)], axis=2)

    xjp, scores = _dense1(x0, W1, b1, W2, b2, a1w, a1b, a2w, a2b, bn=1000)
    scores = scores.reshape(n)

    partials = _make_edge_kernel(n, e)(xjp, scores, rc)

    return _dense2(partials, x0, g1, be1, g2, be2, Wo, bo, bn=1000)


# speedup vs baseline: 1.1291x; 1.1291x over previous
import functools

import jax
import jax.numpy as jnp
from jax import lax
from jax.experimental import pallas as pl
from jax.experimental.pallas import tpu as pltpu
from jax.experimental.pallas import tpu_sc as plsc

D = 128
H = D // 2
NC = 2
NS = 16
L = 16
CHUNK_K = 80
HI32 = -65536
RND = 32768


def _pack2(hi_f32, lo_f32):
    hb = jax.lax.bitcast_convert_type(hi_f32, jnp.int32) + RND
    lb = jax.lax.bitcast_convert_type(lo_f32, jnp.int32) + RND
    return (hb & HI32) | jax.lax.shift_right_logical(lb, 16)


def _dense1_body(x0_ref, w1t_ref, b1_ref, w2t_ref, b2_ref, a1wt_ref, a2wt_ref,
                 a1b_ref, a2b_ref, xjp_ref, sp_ref):
    x = x0_ref[...]
    xi = jnp.dot(x, w1t_ref[...], preferred_element_type=jnp.float32) + b1_ref[...]
    xi = jnp.where(xi > 0, xi, 0.2 * xi)
    xj = jnp.dot(x, w2t_ref[...], preferred_element_type=jnp.float32) + b2_ref[...]
    xj = jnp.where(xj > 0, xj, 0.2 * xj)
    xjp_ref[...] = _pack2(xj[:, :H], xj[:, H:])
    ai = jnp.sum(xi * a1wt_ref[...], axis=1, keepdims=True) + a1b_ref[0, 0]
    aj = jnp.sum(xj * a2wt_ref[...], axis=1, keepdims=True) + a2b_ref[0, 0]
    sp_ref[...] = _pack2(ai, aj)


def _dense1(x0, W1, b1, W2, b2, a1w, a1b, a2w, a2b, bn):
    n = x0.shape[0]
    grid = (n // bn,)
    full = lambda shape: pl.BlockSpec(shape, lambda i: (0, 0))
    return pl.pallas_call(
        _dense1_body,
        grid=grid,
        in_specs=[
            pl.BlockSpec((bn, D), lambda i: (i, 0)),
            full((D, D)), full((1, D)), full((D, D)), full((1, D)),
            full((1, D)), full((1, D)),
            pl.BlockSpec(memory_space=pltpu.SMEM),
            pl.BlockSpec(memory_space=pltpu.SMEM),
        ],
        out_specs=[
            pl.BlockSpec((bn, H), lambda i: (i, 0)),
            pl.BlockSpec((bn, 1), lambda i: (i, 0)),
        ],
        out_shape=[
            jax.ShapeDtypeStruct((n, H), jnp.int32),
            jax.ShapeDtypeStruct((n, 1), jnp.int32),
        ],
    )(x0, W1.T, b1.reshape(1, D), W2.T, b2.reshape(1, D),
      a1w.reshape(1, D), a2w.reshape(1, D),
      a1b.reshape(1, 1), a2b.reshape(1, 1))


def _dense2_body(p_ref, x0_ref, g1_ref, be1_ref, g2_ref, be2_ref, wot_ref,
                 bo_ref, out_ref):
    u = p_ref[0] + p_ref[1] + x0_ref[...]
    mu = jnp.mean(u, axis=1, keepdims=True)
    var = jnp.mean((u - mu) ** 2, axis=1, keepdims=True)
    y = g1_ref[...] * (u - mu) / jnp.sqrt(var + 1e-5) + be1_ref[...]
    v = jnp.dot(y, wot_ref[...], preferred_element_type=jnp.float32) + bo_ref[...] + y
    mu2 = jnp.mean(v, axis=1, keepdims=True)
    var2 = jnp.mean((v - mu2) ** 2, axis=1, keepdims=True)
    out_ref[...] = g2_ref[...] * (v - mu2) / jnp.sqrt(var2 + 1e-5) + be2_ref[...]


def _dense2(partials, x0, g1, be1, g2, be2, Wo, bo, bn):
    n = x0.shape[0]
    grid = (n // bn,)
    full = lambda shape: pl.BlockSpec(shape, lambda i: (0, 0))
    return pl.pallas_call(
        _dense2_body,
        grid=grid,
        in_specs=[
            pl.BlockSpec((2, bn, D), lambda i: (0, i, 0)),
            pl.BlockSpec((bn, D), lambda i: (i, 0)),
            full((1, D)), full((1, D)), full((1, D)), full((1, D)),
            full((D, D)), full((1, D)),
        ],
        out_specs=pl.BlockSpec((bn, D), lambda i: (i, 0)),
        out_shape=jax.ShapeDtypeStruct((n, D), jnp.float32),
    )(partials, x0, g1.reshape(1, D), be1.reshape(1, D), g2.reshape(1, D),
      be2.reshape(1, D), Wo.T, bo.reshape(1, D))


def _make_edge_kernel(n, e):
    nw = NC * NS
    assert e % nw == 0
    e_per_w = e // nw
    K = CHUNK_K
    assert e_per_w % K == 0
    n_chunks = e_per_w // K
    NGB = 3
    NRB = 2
    NIB = 6
    rpt = (n // NS) // 8 * 8
    tail = n - NS * rpt
    assert tail % 8 == 0
    ZQ = 48
    assert rpt % ZQ == 0 and tail <= ZQ and ZQ <= K

    mesh = plsc.VectorSubcoreMesh(core_axis_name="c", subcore_axis_name="s",
                                  num_cores=NC, num_subcores=NS)

    @functools.partial(
        pl.kernel,
        out_type=jax.ShapeDtypeStruct((NC, n, D), jnp.float32),
        mesh=mesh,
        scratch_types=[
            [pltpu.VMEM((2, K), jnp.int32) for _ in range(NIB)],
            [pltpu.VMEM((K, H), jnp.int32) for _ in range(NGB)],
            [pltpu.VMEM((K, D), jnp.float32) for _ in range(NRB)],
            pltpu.VMEM((K,), jnp.float32),
            pltpu.VMEM((n,), jnp.int32),
            pltpu.VMEM_SHARED((n, D), jnp.float32),
            [pltpu.SemaphoreType.DMA for _ in range(NIB)],
            [pltpu.SemaphoreType.DMA for _ in range(NGB)],
            [pltpu.SemaphoreType.DMA for _ in range(NRB)],
            pltpu.SemaphoreType.DMA,
        ],
        compiler_params=pltpu.CompilerParams(needs_layout_passes=False,
                                            use_tc_tiling_on_sc=False),
    )
    def edge_kernel(xjp_hbm, sp_hbm, row_hbm, col_hbm, out_hbm,
                    idx, gbuf, rows, vals_v, spk_v, acc_sh, sI, sg, ss, sz):
        cid = lax.axis_index("c")
        sid = lax.axis_index("s")
        wid = cid * NS + sid

        zeros16 = jnp.zeros((L,), jnp.float32)
        for r in range(ZQ):
            for c in range(D // L):
                rows[0][r, pl.ds(c * L, L)] = zeros16

        def zfire(j, _):
            pltpu.async_copy(rows[0].at[pl.ds(0, ZQ)],
                             acc_sh.at[pl.ds(sid * rpt + j * ZQ, ZQ)], sz)
            return 0

        lax.fori_loop(0, rpt // ZQ, zfire, 0)

        @pl.when(sid == NS - 1)
        def _zero_tail():
            pltpu.async_copy(rows[0].at[pl.ds(0, tail)],
                             acc_sh.at[pl.ds(NS * rpt, tail)], sz)

        pltpu.sync_copy(sp_hbm, spk_v)

        def zdrain(j, _):
            pltpu.make_async_copy(rows[0].at[pl.ds(0, ZQ)],
                                  acc_sh.at[pl.ds(0, ZQ)], sz).wait()
            return 0

        lax.fori_loop(0, rpt // ZQ, zdrain, 0)

        @pl.when(sid == NS - 1)
        def _zero_tail_drain():
            pltpu.make_async_copy(rows[0].at[pl.ds(0, tail)],
                                  acc_sh.at[pl.ds(0, tail)], sz).wait()

        def idx_load(ch, s):
            pltpu.async_copy(row_hbm.at[wid, ch], idx[s].at[pl.ds(0, 1)], sI[s])
            pltpu.async_copy(col_hbm.at[wid, ch], idx[s].at[pl.ds(1, 1)], sI[s])

        def idx_wait(s):
            pltpu.make_async_copy(row_hbm.at[wid, 0], idx[s].at[pl.ds(0, 1)],
                                  sI[s]).wait()
            pltpu.make_async_copy(row_hbm.at[wid, 0], idx[s].at[pl.ds(1, 1)],
                                  sI[s]).wait()

        for s in range(4):
            idx_load(s, s)
        for s in range(2):
            idx_wait(s)

        plsc.subcore_barrier()

        for b in range(2):
            pltpu.async_copy(xjp_hbm.at[idx[b].at[1]], gbuf[b], sg[b])

        def step(j, t):
            p = t % NGB
            p2 = t % NRB
            q = (t + 2) % NGB
            s2 = (t + 2) % NIB
            s4 = (t + 4) % NIB
            static = isinstance(j, int)

            for g in range(K // L):
                r16 = idx[t][0, pl.ds(g * L, L)]
                c16 = idx[t][1, pl.ds(g * L, L)]
                br = plsc.load_gather(spk_v, [r16])
                bc = plsc.load_gather(spk_v, [c16])
                sc = plsc.bitcast(br & HI32, jnp.float32) + \
                    plsc.bitcast(bc << 16, jnp.float32)
                vals_v[pl.ds(g * L, L)] = 1.0 / (1.0 + jnp.exp(-sc))

            pltpu.make_async_copy(xjp_hbm.at[idx[t].at[1]], gbuf[p], sg[p]).wait()

            if not static or j + 2 < n_chunks:
                idx_wait(s2)
                pltpu.async_copy(xjp_hbm.at[idx[s2].at[1]], gbuf[q], sg[q])

            if not static or j >= NRB:
                pltpu.make_async_copy(rows[p2], acc_sh.at[idx[0].at[0]],
                                      ss[p2]).wait()

            if not static or j + 4 < n_chunks:
                idx_load(j + 4, s4)

            @plsc.parallel_loop(0, K, unroll=4)
            def _scale(ei):
                sp = plsc.load_gather(vals_v, [jnp.full((L,), ei, jnp.int32)])
                for w in range(H // L):
                    wv = gbuf[p][ei, pl.ds(w * L, L)]
                    a = plsc.bitcast(wv & HI32, jnp.float32)
                    b = plsc.bitcast(wv << 16, jnp.float32)
                    rows[p2][ei, pl.ds(w * L, L)] = a * sp
                    rows[p2][ei, pl.ds(H + w * L, L)] = b * sp

            pltpu.async_copy(rows[p2], acc_sh.at[idx[t].at[0]], ss[p2], add=True)

        for j in range(NIB):
            step(j, j)

        n_steady = n_chunks // NIB * NIB

        def steady(j6, _):
            for t in range(NIB):
                step(j6 * NIB + t, t)
            return 0

        lax.fori_loop(1, n_steady // NIB, steady, 0)

        for j in range(n_steady, n_chunks):
            step(j, j % NIB)

        for p2 in range(NRB):
            pltpu.make_async_copy(rows[p2], acc_sh.at[idx[0].at[0]],
                                  ss[p2]).wait()
        plsc.subcore_barrier()

        r0 = sid * rpt
        pltpu.sync_copy(acc_sh.at[pl.ds(r0, rpt)],
                        out_hbm.at[cid, pl.ds(r0, rpt)])

        @pl.when(sid == NS - 1)
        def _copy_tail():
            pltpu.sync_copy(acc_sh.at[pl.ds(NS * rpt, tail)],
                            out_hbm.at[cid, pl.ds(NS * rpt, tail)])

    return edge_kernel


def kernel(x0, x1, edge_index, W1, b1, W2, b2, a1w, a1b, a2w, a2b,
           g1, be1, g2, be2, Wo, bo):
    del x1
    n = x0.shape[0]
    e = edge_index.shape[1]
    ei = edge_index.astype(jnp.int32)
    nw = NC * NS
    row3 = ei[0].reshape(nw, -1, 1, CHUNK_K)
    col3 = ei[1].reshape(nw, -1, 1, CHUNK_K)

    xjp, scores = _dense1(x0, W1, b1, W2, b2, a1w, a1b, a2w, a2b, bn=1000)
    scores = scores.reshape(n)

    partials = _make_edge_kernel(n, e)(xjp, scores, row3, col3)

    return _dense2(partials, x0, g1, be1, g2, be2, Wo, bo, bn=1000)


# speedup vs baseline: 1.1590x; 1.0265x over previous
import functools

import jax
import jax.numpy as jnp
from jax import lax
from jax.experimental import pallas as pl
from jax.experimental.pallas import tpu as pltpu
from jax.experimental.pallas import tpu_sc as plsc

D = 128
H = D // 2
NC = 2
NS = 16
L = 16
CHUNK_K = 80
HI32 = -65536
RND = 32768


def _pack2(hi_f32, lo_f32):
    hb = jax.lax.bitcast_convert_type(hi_f32, jnp.int32) + RND
    lb = jax.lax.bitcast_convert_type(lo_f32, jnp.int32) + RND
    return (hb & HI32) | jax.lax.shift_right_logical(lb, 16)


def _dense1_body(x0_ref, w1t_ref, b1_ref, w2t_ref, b2_ref, a1wt_ref, a2wt_ref,
                 a1b_ref, a2b_ref, xjp_ref, sp_ref):
    x = x0_ref[...]
    xi = jnp.dot(x, w1t_ref[...], preferred_element_type=jnp.float32) + b1_ref[...]
    xi = jnp.where(xi > 0, xi, 0.2 * xi)
    xj = jnp.dot(x, w2t_ref[...], preferred_element_type=jnp.float32) + b2_ref[...]
    xj = jnp.where(xj > 0, xj, 0.2 * xj)
    xjp_ref[...] = _pack2(xj[:, :H], xj[:, H:])
    ai = jnp.sum(xi * a1wt_ref[...], axis=1, keepdims=True) + a1b_ref[0, 0]
    aj = jnp.sum(xj * a2wt_ref[...], axis=1, keepdims=True) + a2b_ref[0, 0]
    sp_ref[...] = _pack2(ai, aj)


def _dense1(x0, W1, b1, W2, b2, a1w, a1b, a2w, a2b, bn):
    n = x0.shape[0]
    grid = (n // bn,)
    full = lambda shape: pl.BlockSpec(shape, lambda i: (0, 0))
    return pl.pallas_call(
        _dense1_body,
        grid=grid,
        in_specs=[
            pl.BlockSpec((bn, D), lambda i: (i, 0)),
            full((D, D)), full((1, D)), full((D, D)), full((1, D)),
            full((1, D)), full((1, D)),
            pl.BlockSpec(memory_space=pltpu.SMEM),
            pl.BlockSpec(memory_space=pltpu.SMEM),
        ],
        out_specs=[
            pl.BlockSpec((bn, H), lambda i: (i, 0)),
            pl.BlockSpec((bn, 1), lambda i: (i, 0)),
        ],
        out_shape=[
            jax.ShapeDtypeStruct((n, H), jnp.int32),
            jax.ShapeDtypeStruct((n, 1), jnp.int32),
        ],
    )(x0, W1.T, b1.reshape(1, D), W2.T, b2.reshape(1, D),
      a1w.reshape(1, D), a2w.reshape(1, D),
      a1b.reshape(1, 1), a2b.reshape(1, 1))


def _dense2_body(p_ref, x0_ref, g1_ref, be1_ref, g2_ref, be2_ref, wot_ref,
                 bo_ref, out_ref):
    u = p_ref[0] + p_ref[1] + x0_ref[...]
    mu = jnp.mean(u, axis=1, keepdims=True)
    var = jnp.mean((u - mu) ** 2, axis=1, keepdims=True)
    y = g1_ref[...] * (u - mu) / jnp.sqrt(var + 1e-5) + be1_ref[...]
    v = jnp.dot(y, wot_ref[...], preferred_element_type=jnp.float32) + bo_ref[...] + y
    mu2 = jnp.mean(v, axis=1, keepdims=True)
    var2 = jnp.mean((v - mu2) ** 2, axis=1, keepdims=True)
    out_ref[...] = g2_ref[...] * (v - mu2) / jnp.sqrt(var2 + 1e-5) + be2_ref[...]


def _dense2(partials, x0, g1, be1, g2, be2, Wo, bo, bn):
    n = x0.shape[0]
    grid = (n // bn,)
    full = lambda shape: pl.BlockSpec(shape, lambda i: (0, 0))
    return pl.pallas_call(
        _dense2_body,
        grid=grid,
        in_specs=[
            pl.BlockSpec((2, bn, D), lambda i: (0, i, 0)),
            pl.BlockSpec((bn, D), lambda i: (i, 0)),
            full((1, D)), full((1, D)), full((1, D)), full((1, D)),
            full((D, D)), full((1, D)),
        ],
        out_specs=pl.BlockSpec((bn, D), lambda i: (i, 0)),
        out_shape=jax.ShapeDtypeStruct((n, D), jnp.float32),
    )(partials, x0, g1.reshape(1, D), be1.reshape(1, D), g2.reshape(1, D),
      be2.reshape(1, D), Wo.T, bo.reshape(1, D))


def _make_edge_kernel(n, e):
    nw = NC * NS
    assert e % nw == 0
    e_per_w = e // nw
    K = CHUNK_K
    assert e_per_w % K == 0
    n_chunks = e_per_w // K
    NGB = 3
    NRB = 2
    NIB = 6
    rpt = (n // NS) // 8 * 8
    tail = n - NS * rpt
    assert tail % 8 == 0
    ZQ = 48
    assert rpt % ZQ == 0 and tail <= ZQ and ZQ <= K

    mesh = plsc.VectorSubcoreMesh(core_axis_name="c", subcore_axis_name="s",
                                  num_cores=NC, num_subcores=NS)

    @functools.partial(
        pl.kernel,
        out_type=jax.ShapeDtypeStruct((NC, n, D), jnp.float32),
        mesh=mesh,
        scratch_types=[
            [pltpu.VMEM((2, K), jnp.int32) for _ in range(NIB)],
            [pltpu.VMEM((K, H), jnp.int32) for _ in range(NGB)],
            [pltpu.VMEM((K, D), jnp.float32) for _ in range(NRB)],
            pltpu.VMEM((K,), jnp.float32),
            pltpu.VMEM((n,), jnp.int32),
            pltpu.VMEM_SHARED((n, D), jnp.float32),
            [pltpu.SemaphoreType.DMA for _ in range(NIB)],
            [pltpu.SemaphoreType.DMA for _ in range(NGB)],
            [pltpu.SemaphoreType.DMA for _ in range(NRB)],
            pltpu.SemaphoreType.DMA,
        ],
        compiler_params=pltpu.CompilerParams(needs_layout_passes=False,
                                            use_tc_tiling_on_sc=False),
    )
    def edge_kernel(xjp_hbm, sp_hbm, row_hbm, col_hbm, out_hbm,
                    idx, gbuf, rows, vals_v, spk_v, acc_sh, sI, sg, ss, sz):
        cid = lax.axis_index("c")
        sid = lax.axis_index("s")
        wid = cid * NS + sid

        zeros16 = jnp.zeros((L,), jnp.float32)
        for r in range(ZQ):
            for c in range(D // L):
                rows[0][r, pl.ds(c * L, L)] = zeros16

        def zfire(j, _):
            pltpu.async_copy(rows[0].at[pl.ds(0, ZQ)],
                             acc_sh.at[pl.ds(sid * rpt + j * ZQ, ZQ)], sz)
            return 0

        lax.fori_loop(0, rpt // ZQ, zfire, 0)

        @pl.when(sid == NS - 1)
        def _zero_tail():
            pltpu.async_copy(rows[0].at[pl.ds(0, tail)],
                             acc_sh.at[pl.ds(NS * rpt, tail)], sz)

        pltpu.sync_copy(sp_hbm, spk_v)

        def zdrain(j, _):
            pltpu.make_async_copy(rows[0].at[pl.ds(0, ZQ)],
                                  acc_sh.at[pl.ds(0, ZQ)], sz).wait()
            return 0

        lax.fori_loop(0, rpt // ZQ, zdrain, 0)

        @pl.when(sid == NS - 1)
        def _zero_tail_drain():
            pltpu.make_async_copy(rows[0].at[pl.ds(0, tail)],
                                  acc_sh.at[pl.ds(0, tail)], sz).wait()

        def idx_load(ch, s):
            pltpu.async_copy(row_hbm.at[wid, ch], idx[s].at[pl.ds(0, 1)], sI[s])
            pltpu.async_copy(col_hbm.at[wid, ch], idx[s].at[pl.ds(1, 1)], sI[s])

        def idx_wait(s):
            pltpu.make_async_copy(row_hbm.at[wid, 0], idx[s].at[pl.ds(0, 1)],
                                  sI[s]).wait()
            pltpu.make_async_copy(row_hbm.at[wid, 0], idx[s].at[pl.ds(1, 1)],
                                  sI[s]).wait()

        for s in range(4):
            idx_load(s, s)
        for s in range(2):
            idx_wait(s)

        plsc.subcore_barrier()

        for b in range(2):
            pltpu.async_copy(xjp_hbm.at[idx[b].at[1]], gbuf[b], sg[b])

        def step(j, t):
            p = t % NGB
            p2 = t % NRB
            q = (t + 2) % NGB
            s2 = (t + 2) % NIB
            s4 = (t + 4) % NIB
            static = isinstance(j, int)

            for g in range(K // L):
                r16 = idx[t][0, pl.ds(g * L, L)]
                c16 = idx[t][1, pl.ds(g * L, L)]
                br = plsc.load_gather(spk_v, [r16])
                bc = plsc.load_gather(spk_v, [c16])
                sc = plsc.bitcast(br & HI32, jnp.float32) + \
                    plsc.bitcast(bc << 16, jnp.float32)
                vals_v[pl.ds(g * L, L)] = 1.0 / (1.0 + jnp.exp(-sc))

            pltpu.make_async_copy(xjp_hbm.at[idx[t].at[1]], gbuf[p], sg[p]).wait()

            if not static or j + 2 < n_chunks:
                idx_wait(s2)
                pltpu.async_copy(xjp_hbm.at[idx[s2].at[1]], gbuf[q], sg[q])

            if not static or j >= NRB:
                pltpu.make_async_copy(rows[p2], acc_sh.at[idx[0].at[0]],
                                      ss[p2]).wait()

            if not static or j + 4 < n_chunks:
                idx_load(j + 4, s4)

            @plsc.parallel_loop(0, K, unroll=8)
            def _scale(ei):
                sp = plsc.load_gather(vals_v, [jnp.full((L,), ei, jnp.int32)])
                for w in range(H // L):
                    wv = gbuf[p][ei, pl.ds(w * L, L)]
                    a = plsc.bitcast(wv & HI32, jnp.float32)
                    b = plsc.bitcast(wv << 16, jnp.float32)
                    rows[p2][ei, pl.ds(w * L, L)] = a * sp
                    rows[p2][ei, pl.ds(H + w * L, L)] = b * sp

            pltpu.async_copy(rows[p2], acc_sh.at[idx[t].at[0]], ss[p2], add=True)

        for j in range(NIB):
            step(j, j)

        n_steady = n_chunks // NIB * NIB

        def steady(j6, _):
            for t in range(NIB):
                step(j6 * NIB + t, t)
            return 0

        lax.fori_loop(1, n_steady // NIB, steady, 0)

        for j in range(n_steady, n_chunks):
            step(j, j % NIB)

        for p2 in range(NRB):
            pltpu.make_async_copy(rows[p2], acc_sh.at[idx[0].at[0]],
                                  ss[p2]).wait()
        plsc.subcore_barrier()

        r0 = sid * rpt
        pltpu.sync_copy(acc_sh.at[pl.ds(r0, rpt)],
                        out_hbm.at[cid, pl.ds(r0, rpt)])

        @pl.when(sid == NS - 1)
        def _copy_tail():
            pltpu.sync_copy(acc_sh.at[pl.ds(NS * rpt, tail)],
                            out_hbm.at[cid, pl.ds(NS * rpt, tail)])

    return edge_kernel


def kernel(x0, x1, edge_index, W1, b1, W2, b2, a1w, a1b, a2w, a2b,
           g1, be1, g2, be2, Wo, bo):
    del x1
    n = x0.shape[0]
    e = edge_index.shape[1]
    ei = edge_index.astype(jnp.int32)
    nw = NC * NS
    row3 = ei[0].reshape(nw, -1, 1, CHUNK_K)
    col3 = ei[1].reshape(nw, -1, 1, CHUNK_K)

    xjp, scores = _dense1(x0, W1, b1, W2, b2, a1w, a1b, a2w, a2b, bn=2000)
    scores = scores.reshape(n)

    partials = _make_edge_kernel(n, e)(xjp, scores, row3, col3)

    return _dense2(partials, x0, g1, be1, g2, be2, Wo, bo, bn=2000)


# speedup vs baseline: 1.2043x; 1.0391x over previous
import functools

import jax
import jax.numpy as jnp
from jax import lax
from jax.experimental import pallas as pl
from jax.experimental.pallas import tpu as pltpu
from jax.experimental.pallas import tpu_sc as plsc

D = 128
H = D // 2
NC = 2
NS = 16
L = 16
CHUNK_K = 80
HI32 = -65536
RND = 32768


def _pack2(hi_f32, lo_f32):
    hb = jax.lax.bitcast_convert_type(hi_f32, jnp.int32) + RND
    lb = jax.lax.bitcast_convert_type(lo_f32, jnp.int32) + RND
    return (hb & HI32) | jax.lax.shift_right_logical(lb, 16)


def _dense1_body(x0_ref, w1t_ref, b1_ref, w2t_ref, b2_ref, a1wt_ref, a2wt_ref,
                 a1b_ref, a2b_ref, xjp_ref, sp_ref):
    x = x0_ref[...]
    xi = jnp.dot(x, w1t_ref[...], preferred_element_type=jnp.float32) + b1_ref[...]
    xi = jnp.where(xi > 0, xi, 0.2 * xi)
    xj = jnp.dot(x, w2t_ref[...], preferred_element_type=jnp.float32) + b2_ref[...]
    xj = jnp.where(xj > 0, xj, 0.2 * xj)
    xjp_ref[...] = _pack2(xj[:, :H], xj[:, H:])
    ai = jnp.sum(xi * a1wt_ref[...], axis=1, keepdims=True) + a1b_ref[0, 0]
    aj = jnp.sum(xj * a2wt_ref[...], axis=1, keepdims=True) + a2b_ref[0, 0]
    sp_ref[...] = _pack2(ai, aj)


def _dense1(x0, W1, b1, W2, b2, a1w, a1b, a2w, a2b, bn):
    n = x0.shape[0]
    grid = (n // bn,)
    full = lambda shape: pl.BlockSpec(shape, lambda i: (0, 0))
    return pl.pallas_call(
        _dense1_body,
        grid=grid,
        in_specs=[
            pl.BlockSpec((bn, D), lambda i: (i, 0)),
            full((D, D)), full((1, D)), full((D, D)), full((1, D)),
            full((1, D)), full((1, D)),
            pl.BlockSpec(memory_space=pltpu.SMEM),
            pl.BlockSpec(memory_space=pltpu.SMEM),
        ],
        out_specs=[
            pl.BlockSpec((bn, H), lambda i: (i, 0)),
            pl.BlockSpec((bn, 1), lambda i: (i, 0)),
        ],
        out_shape=[
            jax.ShapeDtypeStruct((n, H), jnp.int32),
            jax.ShapeDtypeStruct((n, 1), jnp.int32),
        ],
    )(x0, W1.T, b1.reshape(1, D), W2.T, b2.reshape(1, D),
      a1w.reshape(1, D), a2w.reshape(1, D),
      a1b.reshape(1, 1), a2b.reshape(1, 1))


def _dense2_body(p_ref, x0_ref, g1_ref, be1_ref, g2_ref, be2_ref, wot_ref,
                 bo_ref, out_ref):
    u = p_ref[0] + p_ref[1] + x0_ref[...]
    mu = jnp.mean(u, axis=1, keepdims=True)
    var = jnp.mean((u - mu) ** 2, axis=1, keepdims=True)
    y = g1_ref[...] * (u - mu) / jnp.sqrt(var + 1e-5) + be1_ref[...]
    v = jnp.dot(y, wot_ref[...], preferred_element_type=jnp.float32) + bo_ref[...] + y
    mu2 = jnp.mean(v, axis=1, keepdims=True)
    var2 = jnp.mean((v - mu2) ** 2, axis=1, keepdims=True)
    out_ref[...] = g2_ref[...] * (v - mu2) / jnp.sqrt(var2 + 1e-5) + be2_ref[...]


def _dense2(partials, x0, g1, be1, g2, be2, Wo, bo, bn):
    n = x0.shape[0]
    grid = (n // bn,)
    full = lambda shape: pl.BlockSpec(shape, lambda i: (0, 0))
    return pl.pallas_call(
        _dense2_body,
        grid=grid,
        in_specs=[
            pl.BlockSpec((2, bn, D), lambda i: (0, i, 0)),
            pl.BlockSpec((bn, D), lambda i: (i, 0)),
            full((1, D)), full((1, D)), full((1, D)), full((1, D)),
            full((D, D)), full((1, D)),
        ],
        out_specs=pl.BlockSpec((bn, D), lambda i: (i, 0)),
        out_shape=jax.ShapeDtypeStruct((n, D), jnp.float32),
    )(partials, x0, g1.reshape(1, D), be1.reshape(1, D), g2.reshape(1, D),
      be2.reshape(1, D), Wo.T, bo.reshape(1, D))


def _make_edge_kernel(n, e):
    nw = NC * NS
    assert e % nw == 0
    e_per_w = e // nw
    K = CHUNK_K
    assert e_per_w % K == 0
    n_chunks = e_per_w // K
    NGB = 3
    NRB = 2
    NIB = 6
    rpt = (n // NS) // 8 * 8
    tail = n - NS * rpt
    assert tail % 8 == 0
    ZQ = 48
    assert rpt % ZQ == 0 and tail <= ZQ and ZQ <= K

    mesh = plsc.VectorSubcoreMesh(core_axis_name="c", subcore_axis_name="s",
                                  num_cores=NC, num_subcores=NS)

    @functools.partial(
        pl.kernel,
        out_type=jax.ShapeDtypeStruct((NC, n, D), jnp.float32),
        mesh=mesh,
        scratch_types=[
            [pltpu.VMEM((2, K), jnp.int32) for _ in range(NIB)],
            [pltpu.VMEM((K, H), jnp.int32) for _ in range(NGB)],
            [pltpu.VMEM((K, D), jnp.float32) for _ in range(NRB)],
            pltpu.VMEM((K,), jnp.float32),
            pltpu.VMEM((n,), jnp.int32),
            pltpu.VMEM_SHARED((n, D), jnp.float32),
            [pltpu.SemaphoreType.DMA for _ in range(NIB)],
            [pltpu.SemaphoreType.DMA for _ in range(NGB)],
            [pltpu.SemaphoreType.DMA for _ in range(NRB)],
            pltpu.SemaphoreType.DMA,
        ],
        compiler_params=pltpu.CompilerParams(needs_layout_passes=False,
                                            use_tc_tiling_on_sc=False),
    )
    def edge_kernel(xjp_hbm, sp_hbm, row_hbm, col_hbm, out_hbm,
                    idx, gbuf, rows, vals_v, spk_v, acc_sh, sI, sg, ss, sz):
        cid = lax.axis_index("c")
        sid = lax.axis_index("s")
        wid = cid * NS + sid

        zeros16 = jnp.zeros((L,), jnp.float32)
        for r in range(ZQ):
            for c in range(D // L):
                rows[0][r, pl.ds(c * L, L)] = zeros16

        def zfire(j, _):
            pltpu.async_copy(rows[0].at[pl.ds(0, ZQ)],
                             acc_sh.at[pl.ds(sid * rpt + j * ZQ, ZQ)], sz)
            return 0

        lax.fori_loop(0, rpt // ZQ, zfire, 0)

        @pl.when(sid == NS - 1)
        def _zero_tail():
            pltpu.async_copy(rows[0].at[pl.ds(0, tail)],
                             acc_sh.at[pl.ds(NS * rpt, tail)], sz)

        pltpu.sync_copy(sp_hbm, spk_v)

        def zdrain(j, _):
            pltpu.make_async_copy(rows[0].at[pl.ds(0, ZQ)],
                                  acc_sh.at[pl.ds(0, ZQ)], sz).wait()
            return 0

        lax.fori_loop(0, rpt // ZQ, zdrain, 0)

        @pl.when(sid == NS - 1)
        def _zero_tail_drain():
            pltpu.make_async_copy(rows[0].at[pl.ds(0, tail)],
                                  acc_sh.at[pl.ds(0, tail)], sz).wait()

        def idx_load(ch, s):
            pltpu.async_copy(row_hbm.at[wid, ch], idx[s].at[pl.ds(0, 1)], sI[s])
            pltpu.async_copy(col_hbm.at[wid, ch], idx[s].at[pl.ds(1, 1)], sI[s])

        def idx_wait(s):
            pltpu.make_async_copy(row_hbm.at[wid, 0], idx[s].at[pl.ds(0, 1)],
                                  sI[s]).wait()
            pltpu.make_async_copy(row_hbm.at[wid, 0], idx[s].at[pl.ds(1, 1)],
                                  sI[s]).wait()

        for s in range(4):
            idx_load(s, s)
        for s in range(2):
            idx_wait(s)

        plsc.subcore_barrier()

        for b in range(2):
            pltpu.async_copy(xjp_hbm.at[idx[b].at[1]], gbuf[b], sg[b])

        def step(j, t):
            p = t % NGB
            p2 = t % NRB
            q = (t + 2) % NGB
            s2 = (t + 2) % NIB
            s4 = (t + 4) % NIB
            static = isinstance(j, int)

            for g in range(K // L):
                r16 = idx[t][0, pl.ds(g * L, L)]
                c16 = idx[t][1, pl.ds(g * L, L)]
                br = plsc.load_gather(spk_v, [r16])
                bc = plsc.load_gather(spk_v, [c16])
                sc = plsc.bitcast(br & HI32, jnp.float32) + \
                    plsc.bitcast(bc << 16, jnp.float32)
                vals_v[pl.ds(g * L, L)] = 1.0 / (1.0 + jnp.exp(-sc))

            pltpu.make_async_copy(xjp_hbm.at[idx[t].at[1]], gbuf[p], sg[p]).wait()

            if not static or j + 2 < n_chunks:
                idx_wait(s2)
                pltpu.async_copy(xjp_hbm.at[idx[s2].at[1]], gbuf[q], sg[q])

            if not static or j >= NRB:
                pltpu.make_async_copy(rows[p2], acc_sh.at[idx[0].at[0]],
                                      ss[p2]).wait()

            if not static or j + 4 < n_chunks:
                idx_load(j + 4, s4)

            @plsc.parallel_loop(0, K, unroll=16)
            def _scale(ei):
                sp = plsc.load_gather(vals_v, [jnp.full((L,), ei, jnp.int32)])
                for w in range(H // L):
                    wv = gbuf[p][ei, pl.ds(w * L, L)]
                    a = plsc.bitcast(wv & HI32, jnp.float32)
                    b = plsc.bitcast(wv << 16, jnp.float32)
                    rows[p2][ei, pl.ds(w * L, L)] = a * sp
                    rows[p2][ei, pl.ds(H + w * L, L)] = b * sp

            pltpu.async_copy(rows[p2], acc_sh.at[idx[t].at[0]], ss[p2], add=True)

        for j in range(NIB):
            step(j, j)

        n_steady = n_chunks // NIB * NIB

        def steady(j6, _):
            for t in range(NIB):
                step(j6 * NIB + t, t)
            return 0

        lax.fori_loop(1, n_steady // NIB, steady, 0)

        for j in range(n_steady, n_chunks):
            step(j, j % NIB)

        for p2 in range(NRB):
            pltpu.make_async_copy(rows[p2], acc_sh.at[idx[0].at[0]],
                                  ss[p2]).wait()
        plsc.subcore_barrier()

        r0 = sid * rpt
        pltpu.sync_copy(acc_sh.at[pl.ds(r0, rpt)],
                        out_hbm.at[cid, pl.ds(r0, rpt)])

        @pl.when(sid == NS - 1)
        def _copy_tail():
            pltpu.sync_copy(acc_sh.at[pl.ds(NS * rpt, tail)],
                            out_hbm.at[cid, pl.ds(NS * rpt, tail)])

    return edge_kernel


def kernel(x0, x1, edge_index, W1, b1, W2, b2, a1w, a1b, a2w, a2b,
           g1, be1, g2, be2, Wo, bo):
    del x1
    n = x0.shape[0]
    e = edge_index.shape[1]
    ei = edge_index.astype(jnp.int32)
    nw = NC * NS
    row3 = ei[0].reshape(nw, -1, 1, CHUNK_K)
    col3 = ei[1].reshape(nw, -1, 1, CHUNK_K)

    xjp, scores = _dense1(x0, W1, b1, W2, b2, a1w, a1b, a2w, a2b, bn=2000)
    scores = scores.reshape(n)

    partials = _make_edge_kernel(n, e)(xjp, scores, row3, col3)

    return _dense2(partials, x0, g1, be1, g2, be2, Wo, bo, bn=2000)
